# Initial kernel scaffold; baseline (speedup 1.0000x reference)
import jax
import jax.numpy as jnp
from jax.experimental import pallas as pl


def kernel(inputs, token_table, pos_table):
    raise NotImplementedError("write your pallas kernel here")


# speedup vs baseline: 1.2514x; 1.2514x over previous
import functools
import math

import jax
import jax.numpy as jnp
from jax import lax
from jax.experimental import pallas as pl
from jax.experimental.pallas import tpu as pltpu
from jax.experimental.pallas import tpu_sc as plsc

D = 128
SEQ = 2048
NB = 4 * SEQ
L = 16
NC = 2
NS = 16
NW = NC * NS
BPW = NB // NW
SCALE = math.sqrt(float(D))

_mesh = plsc.VectorSubcoreMesh(core_axis_name="c", subcore_axis_name="s")


@functools.partial(
    pl.kernel,
    mesh=_mesh,
    out_type=jax.ShapeDtypeStruct((NB, D), jnp.float32),
    scratch_types=[
        pltpu.VMEM((BPW,), jnp.int32),
        pltpu.VMEM((BPW, D), jnp.float32),
        pltpu.VMEM((BPW, D), jnp.float32),
        pltpu.VMEM((BPW,), jnp.float32),
        pltpu.SemaphoreType.DMA,
    ],
)
def _embed_sc(idx_hbm, tok_hbm, pos_hbm, out_hbm, idx_v, rows_v, pos_v,
              mask_v, sem):
    wid = lax.axis_index("s") * NC + lax.axis_index("c")
    base = wid * BPW
    pos_base = lax.rem(base, SEQ)

    pltpu.sync_copy(idx_hbm.at[pl.ds(base, BPW)], idx_v)
    gather = pltpu.async_copy(tok_hbm.at[idx_v], rows_v, sem)
    pltpu.sync_copy(pos_hbm.at[pl.ds(pos_base, BPW)], pos_v)

    for c in range(BPW // L):
        iv = idx_v[pl.ds(c * L, L)]
        mask_v[pl.ds(c * L, L)] = jnp.where(iv != 0, 1.0, 0.0).astype(
            jnp.float32)

    gather.wait()

    def body(g, carry):
        mv = mask_v[pl.ds(g * L, L)]
        for j in range(L):
            mb = mv[j]
            ms = mb * SCALE
            r = g * L + j
            for c in range(D // L):
                sl = pl.ds(c * L, L)
                t = rows_v[r, sl]
                p = pos_v[r, sl]
                rows_v[r, sl] = t * ms + p * mb
        return carry

    lax.fori_loop(0, BPW // L, body, 0)

    pltpu.sync_copy(rows_v, out_hbm.at[pl.ds(base, BPW)])


def kernel(inputs, token_table, pos_table):
    flat_idx = inputs.reshape(NB).astype(jnp.int32)
    out = _embed_sc(flat_idx, token_table, pos_table)
    return out.reshape(inputs.shape[0], inputs.shape[1], D)
